# baseline (speedup 1.0000x reference)
import functools

import jax
import jax.numpy as jnp
import numpy as np
from jax import lax
from jax.experimental import pallas as pl
from jax.experimental.pallas import tpu as pltpu
from jax.experimental.pallas import tpu_sc as plsc

D_MODEL = 256
N_HEADS = 8
N_LEVELS = 4
N_POINTS = 4
N_LAYERS = 6
D_FFN = 1024
SHAPES = [(64, 64), (32, 32), (16, 16), (8, 8)]
LQ = sum(h * w for h, w in SHAPES)
B = 2
DH = D_MODEL // N_HEADS
NPT = N_LEVELS * N_POINTS

NB = 10
T = LQ // NB

N_TILES = 32
TILES_PER_B = N_TILES // B
QPT = LQ // TILES_PER_B
CQ = 5
NCH = QPT // CQ
CHQH = CQ * N_HEADS
ROWS = CHQH * 64

_DH_ORDER = np.concatenate([np.arange(0, DH, 2), np.arange(1, DH, 2)])
_ATTN_PERM = (np.arange(N_HEADS)[:, None] * DH + _DH_ORDER[None, :]).reshape(-1)

_LVL = np.repeat(np.arange(N_LEVELS), N_POINTS)
_W = np.array([s[1] for s in SHAPES], np.float32)[_LVL]
_H = np.array([s[0] for s in SHAPES], np.float32)[_LVL]
_LS = np.array([0, 4096, 5120, 5376], np.int32)[_LVL]


def _lane_const_f(vals):
    return jnp.asarray(vals, jnp.float32)


def _lane_const_i(vals):
    return jnp.asarray(vals, jnp.int32)


def _proj_body(src_ref, pos_ref, rp_ref, wv_ref, bv_ref, wo_ref, bo_ref,
               wa_ref, ba_ref, dims_ref, lin_ref,
               val_ref, sloc_ref, aw_ref, idx_ref, w4_ref):
    s = src_ref[0]
    q = s + pos_ref[0]
    val_ref[0] = (jnp.dot(s.astype(jnp.bfloat16), wv_ref[...].astype(jnp.bfloat16),
                          preferred_element_type=jnp.float32)
                  + bv_ref[0]).astype(jnp.bfloat16)
    sloc = rp_ref[0] + jnp.dot(
        q, wo_ref[...], preferred_element_type=jnp.float32) + bo_ref[0]
    sloc_ref[0] = sloc
    logits = jnp.dot(q, wa_ref[...], preferred_element_type=jnp.float32) + ba_ref[0]
    lg = logits.reshape(T, N_HEADS, NPT)
    m = jnp.max(lg, axis=-1, keepdims=True)
    e = jnp.exp(lg - m)
    aw = (e / jnp.sum(e, axis=-1, keepdims=True)).reshape(T, N_HEADS * NPT)
    aw_ref[0] = aw

    b = pl.program_id(0)
    wf = dims_ref[0, 0][None, :]
    hf = dims_ref[0, 1][None, :]
    wi8 = lin_ref[0, 0][None, :]
    hls = lin_ref[0, 1][None, :]
    px = sloc[:, :128] * wf - 0.5
    py = sloc[:, 128:] * hf - 0.5
    x0f = jnp.floor(px)
    y0f = jnp.floor(py)
    fx = px - x0f
    fy = py - y0f
    x1f = x0f + 1.0
    y1f = y0f + 1.0
    vx0 = (x0f >= 0.0) & (x0f < wf)
    vx1 = (x1f >= 0.0) & (x1f < wf)
    vy0 = (y0f >= 0.0) & (y0f < hf)
    vy1 = (y1f >= 0.0) & (y1f < hf)
    xi0 = jnp.clip(x0f, 0.0, wf - 1.0).astype(jnp.int32) * N_HEADS
    xi1 = jnp.clip(x1f, 0.0, wf - 1.0).astype(jnp.int32) * N_HEADS
    yi0 = jnp.clip(y0f, 0.0, hf - 1.0).astype(jnp.int32)
    yi1 = jnp.clip(y1f, 0.0, hf - 1.0).astype(jnp.int32)
    rbase = b * (LQ * N_HEADS) + hls
    r0 = rbase + yi0 * wi8
    r1 = rbase + yi1 * wi8
    idx_ref[0, 0] = r0 + xi0
    idx_ref[0, 1] = r0 + xi1
    idx_ref[0, 2] = r1 + xi0
    idx_ref[0, 3] = r1 + xi1
    wx0 = 1.0 - fx
    a0 = aw * (1.0 - fy)
    a1 = aw * fy
    zf = jnp.zeros((T, 128), jnp.float32)
    w4_ref[0, 0] = jnp.where(vx0 & vy0, a0 * wx0, zf)
    w4_ref[0, 1] = jnp.where(vx1 & vy0, a0 * fx, zf)
    w4_ref[0, 2] = jnp.where(vx0 & vy1, a1 * wx0, zf)
    w4_ref[0, 3] = jnp.where(vx1 & vy1, a1 * fx, zf)


def _proj_call(src, pos, rp_flat, wv, bv, wo_p, bo_p, wa, ba, dims, lin):
    tok = lambda b, i: (b, i, 0)
    tok4 = lambda b, i: (b, 0, i, 0)
    fixed = lambda b, i: (0, 0)
    fixed3 = lambda b, i: (0, 0, 0)
    return pl.pallas_call(
        _proj_body,
        grid=(B, NB),
        in_specs=[
            pl.BlockSpec((1, T, D_MODEL), tok),
            pl.BlockSpec((1, T, D_MODEL), tok),
            pl.BlockSpec((1, T, D_MODEL), tok),
            pl.BlockSpec((D_MODEL, D_MODEL), fixed),
            pl.BlockSpec((1, D_MODEL), fixed),
            pl.BlockSpec((D_MODEL, D_MODEL), fixed),
            pl.BlockSpec((1, D_MODEL), fixed),
            pl.BlockSpec((D_MODEL, N_HEADS * NPT), fixed),
            pl.BlockSpec((1, N_HEADS * NPT), fixed),
            pl.BlockSpec((1, 2, 128), fixed3),
            pl.BlockSpec((1, 2, 128), fixed3),
        ],
        out_specs=[
            pl.BlockSpec((1, T, D_MODEL), tok),
            pl.BlockSpec((1, T, D_MODEL), tok),
            pl.BlockSpec((1, T, N_HEADS * NPT), tok),
            pl.BlockSpec((1, 4, T, 128), tok4),
            pl.BlockSpec((1, 4, T, 128), tok4),
        ],
        out_shape=[
            jax.ShapeDtypeStruct((B, LQ, D_MODEL), jnp.bfloat16),
            jax.ShapeDtypeStruct((B, LQ, D_MODEL), jnp.float32),
            jax.ShapeDtypeStruct((B, LQ, N_HEADS * NPT), jnp.float32),
            jax.ShapeDtypeStruct((B, 4, LQ, 128), jnp.int32),
            jax.ShapeDtypeStruct((B, 4, LQ, 128), jnp.float32),
        ],
    )(src, pos, rp_flat, wv, bv, wo_p, bo_p, wa, ba, dims, lin)


def _post_body(attn_ref, src_ref, wo_ref, bo_ref, g1_ref, b1_ref,
               w1_ref, bf1_ref, w2_ref, bf2_ref, g2_ref, b2_ref, out_ref):
    a = (jnp.dot(attn_ref[0], wo_ref[...], preferred_element_type=jnp.float32)
         + bo_ref[0] + src_ref[0])
    mu = jnp.mean(a, axis=-1, keepdims=True)
    var = jnp.mean(jnp.square(a - mu), axis=-1, keepdims=True)
    s2 = (a - mu) * jax.lax.rsqrt(var + 1e-5) * g1_ref[0] + b1_ref[0]
    h = jnp.maximum(
        jnp.dot(s2.astype(jnp.bfloat16), w1_ref[...].astype(jnp.bfloat16),
                preferred_element_type=jnp.float32) + bf1_ref[0], 0.0)
    f = (jnp.dot(h.astype(jnp.bfloat16), w2_ref[...].astype(jnp.bfloat16),
                 preferred_element_type=jnp.float32)
         + bf2_ref[0] + s2)
    mu2 = jnp.mean(f, axis=-1, keepdims=True)
    var2 = jnp.mean(jnp.square(f - mu2), axis=-1, keepdims=True)
    out_ref[0] = (f - mu2) * jax.lax.rsqrt(var2 + 1e-5) * g2_ref[0] + b2_ref[0]


def _post_call(attn, src, wo, bo, g1, b1, w1, bf1, w2, bf2, g2, b2):
    tok = lambda b, i: (b, i, 0)
    fixed = lambda b, i: (0, 0)
    return pl.pallas_call(
        _post_body,
        grid=(B, NB),
        in_specs=[
            pl.BlockSpec((1, T, D_MODEL), tok),
            pl.BlockSpec((1, T, D_MODEL), tok),
            pl.BlockSpec((D_MODEL, D_MODEL), fixed),
            pl.BlockSpec((1, D_MODEL), fixed),
            pl.BlockSpec((1, D_MODEL), fixed),
            pl.BlockSpec((1, D_MODEL), fixed),
            pl.BlockSpec((D_MODEL, D_FFN), fixed),
            pl.BlockSpec((1, D_FFN), fixed),
            pl.BlockSpec((D_FFN, D_MODEL), fixed),
            pl.BlockSpec((1, D_MODEL), fixed),
            pl.BlockSpec((1, D_MODEL), fixed),
            pl.BlockSpec((1, D_MODEL), fixed),
        ],
        out_specs=pl.BlockSpec((1, T, D_MODEL), tok),
        out_shape=jax.ShapeDtypeStruct((B, LQ, D_MODEL), jnp.float32),
    )(attn, src, wo, bo, g1, b1, w1, bf1, w2, bf2, g2, b2)


def _sample_body(val_hbm, idx_hbm, w_hbm, out_hbm,
                 idx_v, w_v, rows_v, out_v,
                 sem_in0, sem_in1, sem_g0, sem_g1, sem_o0, sem_o1):
    sem_in = (sem_in0, sem_in1)
    sem_g = (sem_g0, sem_g1)
    sem_o = (sem_o0, sem_o1)
    wid = lax.axis_index("s") * 2 + lax.axis_index("c")
    b = wid % 2
    q0 = (wid // 2) * QPT

    zero = jnp.zeros((16,), jnp.float32)

    CSEG = CQ * 128

    def in_start(g, p):
        q = q0 + g * CQ
        so = (g & 3) * ROWS
        for c in range(4):
            off = ((b * 4 + c) * LQ + q) * 128
            pltpu.async_copy(idx_hbm.at[pl.ds(off, CSEG)],
                             idx_v.at[pl.ds(so + c * CSEG, CSEG)], sem_in[p])
            pltpu.async_copy(w_hbm.at[pl.ds(off, CSEG)],
                             w_v.at[pl.ds(so + c * CSEG, CSEG)], sem_in[p])

    def in_wait(p):
        for c in range(4):
            pltpu.make_async_copy(idx_hbm.at[pl.ds(0, CSEG)],
                                  idx_v.at[pl.ds(c * CSEG, CSEG)],
                                  sem_in[p]).wait()
            pltpu.make_async_copy(w_hbm.at[pl.ds(0, CSEG)],
                                  w_v.at[pl.ds(c * CSEG, CSEG)],
                                  sem_in[p]).wait()

    def fire(g, p):
        so = (g & 3) * ROWS
        pltpu.async_copy(
            val_hbm.at[idx_v.at[pl.ds(so, ROWS)]], rows_v.at[p], sem_g[p])

    def gwait(p):
        pltpu.make_async_copy(
            val_hbm.at[idx_v.at[pl.ds(0, ROWS)]], rows_v.at[p],
            sem_g[p]).wait()

    def accum(g, p):
        gwait(p)
        so = (g & 3) * ROWS

        def acc_qh(qh, carry):
            base = (qh // N_HEADS) * 128 + (qh % N_HEADS) * 16
            a0 = zero
            a1 = zero
            for c in range(4):
                wvec = w_v[pl.ds(so + base + c * 640, 16)]
                for j in range(16):
                    r = base + c * 640 + j
                    w = wvec[j]
                    lo = rows_v[p, r, pl.ds(0, 16)].astype(jnp.float32)
                    hi = rows_v[p, r, pl.ds(16, 16)].astype(jnp.float32)
                    a0 = a0 + w * lo
                    a1 = a1 + w * hi
            out_v[p, pl.ds(qh * DH, 16)] = a0
            out_v[p, pl.ds(qh * DH + 16, 16)] = a1
            return carry

        lax.fori_loop(0, CHQH, acc_qh, 0)
        q = q0 + g * CQ
        soff = (b * LQ + q) * D_MODEL
        pltpu.async_copy(out_v.at[p], out_hbm.at[pl.ds(soff, CQ * D_MODEL)],
                         sem_o[p])

    def out_wait(p):
        pltpu.make_async_copy(out_v.at[p], out_hbm.at[pl.ds(0, CQ * D_MODEL)],
                              sem_o[p]).wait()

    in_start(0, 0)
    in_wait(0)
    fire(0, 0)
    in_start(1, 1)

    def body(i, carry):
        g0 = 2 * i
        g1 = g0 + 1
        in_wait(1)
        fire(g1, 1)
        in_start(jnp.minimum(g0 + 2, NCH - 2), 0)

        @pl.when(i > 0)
        def _():
            out_wait(0)

        accum(g0, 0)
        in_wait(0)
        fire(jnp.minimum(g0 + 2, NCH - 2), 0)
        in_start(jnp.minimum(g1 + 2, NCH - 1), 1)

        @pl.when(i > 0)
        def _():
            out_wait(1)

        accum(g1, 1)
        return carry

    lax.fori_loop(0, NCH // 2, body, 0)
    gwait(0)
    in_wait(1)
    out_wait(0)
    out_wait(1)


@functools.cache
def _make_sample_call():
    return pl.kernel(
        _sample_body,
        out_type=jax.ShapeDtypeStruct((B * LQ * D_MODEL,), jnp.float32),
        mesh=plsc.VectorSubcoreMesh(
            core_axis_name="c", subcore_axis_name="s",
            num_cores=2, num_subcores=16),
        compiler_params=pltpu.CompilerParams(use_tc_tiling_on_sc=False),
        scratch_types=[
            pltpu.VMEM((4 * ROWS,), jnp.int32),
            pltpu.VMEM((4 * ROWS,), jnp.float32),
            pltpu.VMEM((2, ROWS, DH), jnp.bfloat16),
            pltpu.VMEM((2, CQ * D_MODEL), jnp.float32),
            pltpu.SemaphoreType.DMA,
            pltpu.SemaphoreType.DMA,
            pltpu.SemaphoreType.DMA,
            pltpu.SemaphoreType.DMA,
            pltpu.SemaphoreType.DMA,
            pltpu.SemaphoreType.DMA,
        ],
    )


def _sample_call(val, idx, w):
    return _make_sample_call()(val, idx, w)


def _ref_points_flat(valid_ratios):
    ref_list = []
    for lvl, (H_, W_) in enumerate(SHAPES):
        ry, rx = jnp.meshgrid(jnp.linspace(0.5, H_ - 0.5, H_),
                              jnp.linspace(0.5, W_ - 0.5, W_), indexing='ij')
        ry = ry.reshape(-1)[None] / (valid_ratios[:, None, lvl, 1] * H_)
        rx = rx.reshape(-1)[None] / (valid_ratios[:, None, lvl, 0] * W_)
        ref_list.append(jnp.stack((rx, ry), -1))
    rp = jnp.concatenate(ref_list, 1)
    rp = rp[:, :, None] * valid_ratios[:, None]
    rp = rp.transpose(0, 1, 3, 2)
    rp = jnp.repeat(rp, N_POINTS, axis=-1)
    rp = jnp.broadcast_to(rp[:, :, :, None], (B, LQ, 2, N_HEADS, NPT))
    return rp.reshape(B, LQ, D_MODEL)


def kernel(src, spatial_shapes, level_start_index, valid_ratios, pos, params):
    rp_flat = _ref_points_flat(valid_ratios)

    inv = np.zeros((2, N_HEADS, NPT), np.float32)
    inv[0] = 1.0 / _W
    inv[1] = 1.0 / _H
    inv_flat = jnp.asarray(inv.reshape(D_MODEL))
    wo_p = params['W_off'].reshape(N_LAYERS, D_MODEL, N_HEADS, N_LEVELS, N_POINTS, 2)
    wo_p = wo_p.transpose(0, 1, 5, 2, 3, 4).reshape(N_LAYERS, D_MODEL, D_MODEL) * inv_flat
    bo_p = params['b_off'].reshape(N_LAYERS, N_HEADS, N_LEVELS, N_POINTS, 2)
    bo_p = bo_p.transpose(0, 4, 1, 2, 3).reshape(N_LAYERS, 1, D_MODEL) * inv_flat
    wout_p = params['W_out']

    dims = jnp.asarray(np.stack([np.tile(_W, N_HEADS), np.tile(_H, N_HEADS)])
                       )[None]
    hls = (np.arange(N_HEADS, dtype=np.int32)[:, None]
           + (np.asarray(_LS) * N_HEADS)[None, :]).reshape(-1)
    lin = jnp.asarray(np.stack([np.tile((_W * N_HEADS).astype(np.int32), N_HEADS),
                                hls]))[None]

    out = src
    sl_all, aw_all = [], []
    for lid in range(N_LAYERS):
        p = {k: v[lid] for k, v in params.items()}
        val, sloc_p, aw, idx4, w4 = _proj_call(
            out, pos, rp_flat,
            p['W_value'], p['b_value'][None],
            wo_p[lid], bo_p[lid],
            p['W_attn'], p['b_attn'][None],
            dims, lin)
        attn_flat = _sample_call(
            val.reshape(B * LQ * N_HEADS, DH),
            idx4.reshape(-1), w4.reshape(-1))
        out = _post_call(
            attn_flat.reshape(B, LQ, D_MODEL), out,
            wout_p[lid], p['b_out'][None],
            p['ln1_g'][None], p['ln1_b'][None],
            p['W_ff1'], p['b_ff1'][None],
            p['W_ff2'], p['b_ff2'][None],
            p['ln2_g'][None], p['ln2_b'][None])
        sl_all.append(sloc_p)
        aw_all.append(aw)

    sl = jnp.stack(sl_all, 1).reshape(B, N_LAYERS, LQ, 2, N_HEADS, N_LEVELS, N_POINTS)
    sl = sl.transpose(0, 1, 2, 4, 5, 6, 3)
    aw = jnp.stack(aw_all, 1).reshape(B, N_LAYERS, LQ, N_HEADS, N_LEVELS, N_POINTS)
    return out, sl, aw
